# Initial kernel scaffold; baseline (speedup 1.0000x reference)
import jax
import jax.numpy as jnp
from jax.experimental import pallas as pl


def kernel(x_feat, x_agg, ei_c2a_src, ei_c2a_dst, ei_a2c_src, ei_a2c_dst, enc_W, enc_b, l1_c2a_Wl, l1_c2a_bl, l1_c2a_Wr, l1_a2c_Wl, l1_a2c_bl, l1_a2c_Wr, l2_c2a_Wl, l2_c2a_bl, l2_c2a_Wr, l2_a2c_Wl, l2_a2c_bl, l2_a2c_Wr, lin_W, lin_b):
    raise NotImplementedError("write your pallas kernel here")


# speedup vs baseline: 4.2659x; 4.2659x over previous
import functools

import jax
import jax.numpy as jnp
from jax import lax
from jax.experimental import pallas as pl
from jax.experimental.pallas import tpu as pltpu
from jax.experimental.pallas import tpu_sc as plsc

N_CLIENTS = 10000
N_AGG = 1000
E = 320000
D = 128

NC = 2
NS = 16
NW = NC * NS
EPW = E // NW
CH = 80
NCHUNK = EPW // CH

_mesh = plsc.VectorSubcoreMesh(core_axis_name="c", subcore_axis_name="s")


def _fill_vmem2d(ref, val):
    rows, width = ref.shape
    v = jnp.full((16,), val, jnp.float32)

    def body(i, carry):
        for j in range(width // 16):
            ref[i, pl.ds(j * 16, 16)] = v
        return carry

    lax.fori_loop(0, rows, body, 0)


def _zero_shared(acc, zsrc, sid):
    n = acc.shape[0]
    zr = zsrc.shape[0]
    nwin = (n + zr - 1) // zr
    for w in range(nwin):
        sz = min(zr, n - w * zr)

        @pl.when(sid == (w % NS))
        def _():
            src = zsrc if sz == zr else zsrc.at[pl.ds(0, sz)]
            pltpu.sync_copy(src, acc.at[pl.ds(w * zr, sz)])


def _writeback(acc, out_hbm, bounce, core, sid):
    n = acc.shape[0]
    zr = bounce.shape[0]
    nwin = (n + zr - 1) // zr
    for w in range(nwin):
        sz = min(zr, n - w * zr)

        @pl.when(sid == (w % NS))
        def _():
            dst = bounce if sz == zr else bounce.at[pl.ds(0, sz)]
            pltpu.sync_copy(acc.at[pl.ds(w * zr, sz)], dst)
            pltpu.sync_copy(dst, out_hbm.at[core, pl.ds(w * zr, sz)])


def _edge_pass(table_hbm, src_hbm, dst_hbm, acc, sidx, didx, rows, sem, wid):
    def chunk(k, carry):
        base = wid * EPW + k * CH
        pltpu.sync_copy(src_hbm.at[pl.ds(base, CH)], sidx)
        pltpu.sync_copy(dst_hbm.at[pl.ds(base, CH)], didx)
        pltpu.async_copy(table_hbm.at[sidx], rows, sem).wait()
        pltpu.sync_copy(rows, acc.at[didx], add=True)
        return carry

    lax.fori_loop(0, NCHUNK, chunk, 0)


def _count_pass(dst_hbm, cacc, didx, ones_b, wid):
    def chunk(k, carry):
        base = wid * EPW + k * CH
        pltpu.sync_copy(dst_hbm.at[pl.ds(base, CH)], didx)
        pltpu.sync_copy(ones_b, cacc.at[didx], add=True)
        return carry

    lax.fori_loop(0, NCHUNK, chunk, 0)


@functools.partial(
    pl.kernel,
    out_type=(
        jax.ShapeDtypeStruct((NC, N_AGG, D), jnp.float32),
        jax.ShapeDtypeStruct((NC, N_CLIENTS, D), jnp.float32),
    ),
    mesh=_mesh,
    scratch_types=[
        pltpu.VMEM((CH,), jnp.int32),
        pltpu.VMEM((CH, D), jnp.float32),
        pltpu.VMEM((CH, D), jnp.float32),
        pltpu.VMEM_SHARED((N_AGG, D), jnp.float32),
        pltpu.VMEM_SHARED((N_CLIENTS, D), jnp.float32),
    ],
)
def _sc_counts(c2a_dst, a2c_dst, cntA_hbm, cntC_hbm,
               didx, ones_b, zrows, caccA, caccC):
    core = lax.axis_index("c")
    sid = lax.axis_index("s")
    wid = core * NS + sid

    _fill_vmem2d(ones_b, 1.0)
    _fill_vmem2d(zrows, 0.0)
    _zero_shared(caccA, zrows, sid)
    _zero_shared(caccC, zrows, sid)
    plsc.subcore_barrier()

    _count_pass(c2a_dst, caccA, didx, ones_b, wid)
    _count_pass(a2c_dst, caccC, didx, ones_b, wid)
    plsc.subcore_barrier()

    _writeback(caccA, cntA_hbm, zrows, core, sid)
    _writeback(caccC, cntC_hbm, zrows, core, sid)


@functools.partial(
    pl.kernel,
    out_type=(
        jax.ShapeDtypeStruct((NC, N_AGG, D), jnp.float32),
        jax.ShapeDtypeStruct((NC, N_CLIENTS, D), jnp.float32),
    ),
    mesh=_mesh,
    scratch_types=[
        pltpu.VMEM((CH,), jnp.int32),
        pltpu.VMEM((CH,), jnp.int32),
        pltpu.VMEM((CH, D), jnp.float32),
        pltpu.VMEM_SHARED((N_AGG, D), jnp.float32),
        pltpu.VMEM_SHARED((N_CLIENTS, D), jnp.float32),
        pltpu.SemaphoreType.DMA,
    ],
)
def _sc_layer1(clients_hbm, aggs_hbm, c2a_src, c2a_dst, a2c_src, a2c_dst,
               segA_hbm, segC_hbm,
               sidx, didx, rows, accA, accC, sem):
    core = lax.axis_index("c")
    sid = lax.axis_index("s")
    wid = core * NS + sid

    _fill_vmem2d(rows, 0.0)
    _zero_shared(accA, rows, sid)
    _zero_shared(accC, rows, sid)
    plsc.subcore_barrier()

    _edge_pass(clients_hbm, c2a_src, c2a_dst, accA, sidx, didx, rows,
               sem, wid)
    _edge_pass(aggs_hbm, a2c_src, a2c_dst, accC, sidx, didx, rows,
               sem, wid)
    plsc.subcore_barrier()

    _writeback(accA, segA_hbm, rows, core, sid)
    _writeback(accC, segC_hbm, rows, core, sid)


@functools.partial(
    pl.kernel,
    out_type=jax.ShapeDtypeStruct((NC, N_CLIENTS, D), jnp.float32),
    mesh=_mesh,
    scratch_types=[
        pltpu.VMEM((CH,), jnp.int32),
        pltpu.VMEM((CH,), jnp.int32),
        pltpu.VMEM((CH, D), jnp.float32),
        pltpu.VMEM_SHARED((N_CLIENTS, D), jnp.float32),
        pltpu.SemaphoreType.DMA,
    ],
)
def _sc_layer2(aggs_hbm, a2c_src, a2c_dst, segC_hbm,
               sidx, didx, rows, accC, sem):
    core = lax.axis_index("c")
    sid = lax.axis_index("s")
    wid = core * NS + sid

    _fill_vmem2d(rows, 0.0)
    _zero_shared(accC, rows, sid)
    plsc.subcore_barrier()

    _edge_pass(aggs_hbm, a2c_src, a2c_dst, accC, sidx, didx, rows,
               sem, wid)
    plsc.subcore_barrier()

    _writeback(accC, segC_hbm, rows, core, sid)


def _encoder(x_feat, enc_W, enc_b):
    n = x_feat.shape[0]
    b = 1000

    def body(x_ref, w_ref, b_ref, o_ref):
        o_ref[...] = (jnp.dot(x_ref[...], w_ref[...],
                              preferred_element_type=jnp.float32)
                      + b_ref[...]) * 0.5

    return pl.pallas_call(
        body,
        grid=(n // b,),
        in_specs=[
            pl.BlockSpec((b, D), lambda i: (i, 0)),
            pl.BlockSpec((D, D), lambda i: (0, 0)),
            pl.BlockSpec((1, D), lambda i: (0, 0)),
        ],
        out_specs=pl.BlockSpec((b, D), lambda i: (i, 0)),
        out_shape=jax.ShapeDtypeStruct((n, D), jnp.float32),
    )(x_feat, enc_W, enc_b.reshape(1, D))


def _cnt_reduce(cnt):
    n = cnt.shape[1]
    b = 1000

    def body(c_ref, o_ref):
        o_ref[...] = c_ref[0, :, 0:1] + c_ref[1, :, 0:1]

    return pl.pallas_call(
        body,
        grid=(n // b,),
        in_specs=[pl.BlockSpec((NC, b, D), lambda i: (0, i, 0))],
        out_specs=pl.BlockSpec((b, 1), lambda i: (i, 0)),
        out_shape=jax.ShapeDtypeStruct((n, 1), jnp.float32),
    )(cnt)


def _combine(seg, cnt, xdst, Wl, bl, Wr):
    n = xdst.shape[0]
    b = 1000

    def body(seg_ref, cnt_ref, x_ref, wl_ref, bl_ref, wr_ref, o_ref):
        segs = seg_ref[0] + seg_ref[1]
        mean = segs / jnp.maximum(cnt_ref[...], 1.0)
        y = (jnp.dot(mean, wl_ref[...], preferred_element_type=jnp.float32)
             + bl_ref[...]
             + jnp.dot(x_ref[...], wr_ref[...],
                       preferred_element_type=jnp.float32))
        o_ref[...] = jnp.where(y >= 0.0, y, 0.1 * y)

    return pl.pallas_call(
        body,
        grid=(n // b,),
        in_specs=[
            pl.BlockSpec((NC, b, D), lambda i: (0, i, 0)),
            pl.BlockSpec((b, 1), lambda i: (i, 0)),
            pl.BlockSpec((b, D), lambda i: (i, 0)),
            pl.BlockSpec((D, D), lambda i: (0, 0)),
            pl.BlockSpec((1, D), lambda i: (0, 0)),
            pl.BlockSpec((D, D), lambda i: (0, 0)),
        ],
        out_specs=pl.BlockSpec((b, D), lambda i: (i, 0)),
        out_shape=jax.ShapeDtypeStruct((n, D), jnp.float32),
    )(seg, cnt, xdst, Wl, bl.reshape(1, D), Wr)


def _combine_final(seg, cnt, xdst, Wl, bl, Wr, lin_W, lin_b):
    n = xdst.shape[0]
    b = 1000

    def body(seg_ref, cnt_ref, x_ref, wl_ref, bl_ref, wr_ref, lw_ref, lb_ref,
             o_ref):
        segs = seg_ref[0] + seg_ref[1]
        mean = segs / jnp.maximum(cnt_ref[...], 1.0)
        y = (jnp.dot(mean, wl_ref[...], preferred_element_type=jnp.float32)
             + bl_ref[...]
             + jnp.dot(x_ref[...], wr_ref[...],
                       preferred_element_type=jnp.float32))
        y = jnp.where(y >= 0.0, y, 0.1 * y)
        o_ref[...] = (jnp.dot(y, lw_ref[...],
                              preferred_element_type=jnp.float32)
                      + lb_ref[...])

    return pl.pallas_call(
        body,
        grid=(n // b,),
        in_specs=[
            pl.BlockSpec((NC, b, D), lambda i: (0, i, 0)),
            pl.BlockSpec((b, 1), lambda i: (i, 0)),
            pl.BlockSpec((b, D), lambda i: (i, 0)),
            pl.BlockSpec((D, D), lambda i: (0, 0)),
            pl.BlockSpec((1, D), lambda i: (0, 0)),
            pl.BlockSpec((D, D), lambda i: (0, 0)),
            pl.BlockSpec((D, 1), lambda i: (0, 0)),
            pl.BlockSpec((1, 1), lambda i: (0, 0)),
        ],
        out_specs=pl.BlockSpec((b, 1), lambda i: (i, 0)),
        out_shape=jax.ShapeDtypeStruct((n, 1), jnp.float32),
    )(seg, cnt, xdst, Wl, bl.reshape(1, D), Wr, lin_W,
      lin_b.reshape(1, 1))


def kernel(x_feat, x_agg, ei_c2a_src, ei_c2a_dst, ei_a2c_src, ei_a2c_dst,
           enc_W, enc_b, l1_c2a_Wl, l1_c2a_bl, l1_c2a_Wr, l1_a2c_Wl,
           l1_a2c_bl, l1_a2c_Wr, l2_c2a_Wl, l2_c2a_bl, l2_c2a_Wr, l2_a2c_Wl,
           l2_a2c_bl, l2_a2c_Wr, lin_W, lin_b):
    clients0 = _encoder(x_feat, enc_W, enc_b)
    cntA, cntC = _sc_counts(ei_c2a_dst, ei_a2c_dst)
    segA, segC = _sc_layer1(
        clients0, x_agg, ei_c2a_src, ei_c2a_dst, ei_a2c_src, ei_a2c_dst)
    cntA_r = _cnt_reduce(cntA)
    cntC_r = _cnt_reduce(cntC)
    aggs1 = _combine(segA, cntA_r, x_agg, l1_c2a_Wl, l1_c2a_bl, l1_c2a_Wr)
    clients1 = _combine(segC, cntC_r, clients0, l1_a2c_Wl, l1_a2c_bl,
                        l1_a2c_Wr)
    segC2 = _sc_layer2(aggs1, ei_a2c_src, ei_a2c_dst)
    out = _combine_final(segC2, cntC_r, clients1, l2_a2c_Wl, l2_a2c_bl,
                         l2_a2c_Wr, lin_W, lin_b)
    return out[:, 0]


# speedup vs baseline: 8.2608x; 1.9365x over previous
import functools

import jax
import jax.numpy as jnp
from jax import lax
from jax.experimental import pallas as pl
from jax.experimental.pallas import tpu as pltpu
from jax.experimental.pallas import tpu_sc as plsc

N_CLIENTS = 10000
N_AGG = 1000
E = 320000
D = 128

NC = 2
NS = 16
NW = NC * NS
EPW = E // NW
CH = 80
NCHUNK = EPW // CH
RCH = 25
RND = NCHUNK // RCH

_mesh = plsc.VectorSubcoreMesh(core_axis_name="c", subcore_axis_name="s")


def _fill_vmem2d(ref, val):
    rows, width = ref.shape
    v = jnp.full((16,), val, jnp.float32)

    def body(i, carry):
        for j in range(width // 16):
            ref[i, pl.ds(j * 16, 16)] = v
        return carry

    lax.fori_loop(0, rows, body, 0)


def _zero_shared(acc, zsrc, sid):
    n = acc.shape[0]
    zr = zsrc.shape[0]
    nwin = (n + zr - 1) // zr
    for w in range(nwin):
        sz = min(zr, n - w * zr)

        @pl.when(sid == (w % NS))
        def _():
            src = zsrc if sz == zr else zsrc.at[pl.ds(0, sz)]
            pltpu.sync_copy(src, acc.at[pl.ds(w * zr, sz)])


def _writeback(acc, out_hbm, bounce, core, sid):
    n = acc.shape[0]
    zr = bounce.shape[0]
    nwin = (n + zr - 1) // zr
    for w in range(nwin):
        sz = min(zr, n - w * zr)

        @pl.when(sid == (w % NS))
        def _():
            dst = bounce if sz == zr else bounce.at[pl.ds(0, sz)]
            pltpu.sync_copy(acc.at[pl.ds(w * zr, sz)], dst)
            pltpu.sync_copy(dst, out_hbm.at[core, pl.ds(w * zr, sz)])


def _edge_pass(table_hbm, src_hbm, dst_hbm, acc,
               sidx_a, sidx_b, didx_a, didx_b, rows_a, rows_b,
               semg_a, semg_b, semi_a, semi_b, wid):
    base = wid * EPW

    def _load_idx(c, sv, dv, sem):
        off = base + jnp.minimum(c, NCHUNK - 1) * CH
        pltpu.async_copy(src_hbm.at[pl.ds(off, CH)], sv, sem)
        pltpu.async_copy(dst_hbm.at[pl.ds(off, CH)], dv, sem)

    def _drain_idx(sv, dv, sem):
        pltpu.make_async_copy(src_hbm.at[pl.ds(0, CH)], sv, sem).wait()
        pltpu.make_async_copy(src_hbm.at[pl.ds(0, CH)], dv, sem).wait()

    def _gather(sv, buf, sem):
        pltpu.async_copy(table_hbm.at[sv], buf, sem)

    def _drain_g(buf, sem):
        pltpu.make_async_copy(table_hbm.at[sidx_a], buf, sem).wait()

    def _scatter(dv, buf):
        pltpu.sync_copy(buf, acc.at[dv], add=True)

    _load_idx(0, sidx_a, didx_a, semi_a)
    _drain_idx(sidx_a, didx_a, semi_a)
    _gather(sidx_a, rows_a, semg_a)
    _load_idx(1, sidx_b, didx_b, semi_b)

    def pair(k, carry):
        c0 = 2 * k
        _drain_idx(sidx_b, didx_b, semi_b)
        _gather(sidx_b, rows_b, semg_b)
        _drain_g(rows_a, semg_a)
        _scatter(didx_a, rows_a)
        _load_idx(c0 + 2, sidx_a, didx_a, semi_a)
        _drain_idx(sidx_a, didx_a, semi_a)
        _gather(sidx_a, rows_a, semg_a)
        _drain_g(rows_b, semg_b)
        _scatter(didx_b, rows_b)
        _load_idx(c0 + 3, sidx_b, didx_b, semi_b)
        return carry

    lax.fori_loop(0, (NCHUNK - 1) // 2, pair, 0)
    _drain_idx(sidx_b, didx_b, semi_b)
    _drain_g(rows_a, semg_a)
    _scatter(didx_a, rows_a)


def _count_pass(dst_hbm, cacc, didx_a, didx_b, ones_b, semi_a, semi_b, wid):
    base = wid * EPW

    def _load_idx(c, dv, sem):
        off = base + jnp.minimum(c, NCHUNK - 1) * CH
        pltpu.async_copy(dst_hbm.at[pl.ds(off, CH)], dv, sem)

    def _drain_idx(dv, sem):
        pltpu.make_async_copy(dst_hbm.at[pl.ds(0, CH)], dv, sem).wait()

    _load_idx(0, didx_a, semi_a)
    _drain_idx(didx_a, semi_a)
    _load_idx(1, didx_b, semi_b)

    def pair(k, carry):
        c0 = 2 * k
        pltpu.sync_copy(ones_b, cacc.at[didx_a], add=True)
        _load_idx(c0 + 2, didx_a, semi_a)
        _drain_idx(didx_b, semi_b)
        pltpu.sync_copy(ones_b, cacc.at[didx_b], add=True)
        _load_idx(c0 + 3, didx_b, semi_b)
        _drain_idx(didx_a, semi_a)
        return carry

    lax.fori_loop(0, (NCHUNK - 1) // 2, pair, 0)
    _drain_idx(didx_b, semi_b)
    pltpu.sync_copy(ones_b, cacc.at[didx_a], add=True)


@functools.partial(
    pl.kernel,
    out_type=(
        jax.ShapeDtypeStruct((NC, N_AGG, D), jnp.float32),
        jax.ShapeDtypeStruct((NC, N_CLIENTS, D), jnp.float32),
    ),
    mesh=_mesh,
    scratch_types=[
        pltpu.VMEM((CH,), jnp.int32),
        pltpu.VMEM((CH,), jnp.int32),
        pltpu.VMEM((CH, D), jnp.float32),
        pltpu.VMEM((CH, D), jnp.float32),
        pltpu.VMEM_SHARED((N_AGG, D), jnp.float32),
        pltpu.VMEM_SHARED((N_CLIENTS, D), jnp.float32),
        pltpu.SemaphoreType.DMA,
        pltpu.SemaphoreType.DMA,
    ],
)
def _sc_counts(c2a_dst, a2c_dst, cntA_hbm, cntC_hbm,
               didx_a, didx_b, ones_b, zrows, caccA, caccC,
               semi_a, semi_b):
    core = lax.axis_index("c")
    sid = lax.axis_index("s")
    wid = core * NS + sid

    _fill_vmem2d(ones_b, 1.0)
    _fill_vmem2d(zrows, 0.0)
    _zero_shared(caccA, zrows, sid)
    _zero_shared(caccC, zrows, sid)
    plsc.subcore_barrier()

    _count_pass(c2a_dst, caccA, didx_a, didx_b, ones_b, semi_a, semi_b, wid)
    _count_pass(a2c_dst, caccC, didx_a, didx_b, ones_b, semi_a, semi_b, wid)
    plsc.subcore_barrier()

    _writeback(caccA, cntA_hbm, zrows, core, sid)
    _writeback(caccC, cntC_hbm, zrows, core, sid)


@functools.partial(
    pl.kernel,
    out_type=(
        jax.ShapeDtypeStruct((NC, N_AGG, D), jnp.float32),
        jax.ShapeDtypeStruct((NC, N_CLIENTS, D), jnp.float32),
    ),
    mesh=_mesh,
    scratch_types=[
        pltpu.VMEM((CH,), jnp.int32),
        pltpu.VMEM((CH,), jnp.int32),
        pltpu.VMEM((CH,), jnp.int32),
        pltpu.VMEM((CH,), jnp.int32),
        pltpu.VMEM((CH, D), jnp.float32),
        pltpu.VMEM((CH, D), jnp.float32),
        pltpu.VMEM_SHARED((N_AGG, D), jnp.float32),
        pltpu.VMEM_SHARED((N_CLIENTS, D), jnp.float32),
        pltpu.SemaphoreType.DMA,
        pltpu.SemaphoreType.DMA,
        pltpu.SemaphoreType.DMA,
        pltpu.SemaphoreType.DMA,
    ],
)
def _sc_layer1(clients_hbm, aggs_hbm, c2a_src, c2a_dst, a2c_src, a2c_dst,
               segA_hbm, segC_hbm,
               sidx_a, sidx_b, didx_a, didx_b, rows_a, rows_b, accA, accC,
               semg_a, semg_b, semi_a, semi_b):
    core = lax.axis_index("c")
    sid = lax.axis_index("s")
    wid = core * NS + sid

    _fill_vmem2d(rows_a, 0.0)
    _zero_shared(accA, rows_a, sid)
    _zero_shared(accC, rows_a, sid)
    plsc.subcore_barrier()

    _edge_pass(clients_hbm, c2a_src, c2a_dst, accA, sidx_a, sidx_b,
               didx_a, didx_b, rows_a, rows_b, semg_a, semg_b, semi_a,
               semi_b, wid)
    _edge_pass(aggs_hbm, a2c_src, a2c_dst, accC, sidx_a, sidx_b,
               didx_a, didx_b, rows_a, rows_b, semg_a, semg_b, semi_a,
               semi_b, wid)
    plsc.subcore_barrier()

    _writeback(accA, segA_hbm, rows_a, core, sid)
    _writeback(accC, segC_hbm, rows_a, core, sid)


@functools.partial(
    pl.kernel,
    out_type=jax.ShapeDtypeStruct((NC, N_CLIENTS, D), jnp.float32),
    mesh=_mesh,
    scratch_types=[
        pltpu.VMEM((CH,), jnp.int32),
        pltpu.VMEM((CH,), jnp.int32),
        pltpu.VMEM((CH,), jnp.int32),
        pltpu.VMEM((CH,), jnp.int32),
        pltpu.VMEM((CH, D), jnp.float32),
        pltpu.VMEM((CH, D), jnp.float32),
        pltpu.VMEM_SHARED((N_CLIENTS, D), jnp.float32),
        pltpu.SemaphoreType.DMA,
        pltpu.SemaphoreType.DMA,
        pltpu.SemaphoreType.DMA,
        pltpu.SemaphoreType.DMA,
    ],
)
def _sc_layer2(aggs_hbm, a2c_src, a2c_dst, segC_hbm,
               sidx_a, sidx_b, didx_a, didx_b, rows_a, rows_b, accC,
               semg_a, semg_b, semi_a, semi_b):
    core = lax.axis_index("c")
    sid = lax.axis_index("s")
    wid = core * NS + sid

    _fill_vmem2d(rows_a, 0.0)
    _zero_shared(accC, rows_a, sid)
    plsc.subcore_barrier()

    _edge_pass(aggs_hbm, a2c_src, a2c_dst, accC, sidx_a, sidx_b,
               didx_a, didx_b, rows_a, rows_b, semg_a, semg_b, semi_a,
               semi_b, wid)
    plsc.subcore_barrier()

    _writeback(accC, segC_hbm, rows_a, core, sid)


def _encoder(x_feat, enc_W, enc_b):
    n = x_feat.shape[0]
    b = 1000

    def body(x_ref, w_ref, b_ref, o_ref):
        o_ref[...] = (jnp.dot(x_ref[...], w_ref[...],
                              preferred_element_type=jnp.float32)
                      + b_ref[...]) * 0.5

    return pl.pallas_call(
        body,
        grid=(n // b,),
        in_specs=[
            pl.BlockSpec((b, D), lambda i: (i, 0)),
            pl.BlockSpec((D, D), lambda i: (0, 0)),
            pl.BlockSpec((1, D), lambda i: (0, 0)),
        ],
        out_specs=pl.BlockSpec((b, D), lambda i: (i, 0)),
        out_shape=jax.ShapeDtypeStruct((n, D), jnp.float32),
    )(x_feat, enc_W, enc_b.reshape(1, D))


def _cnt_reduce(cnt):
    n = cnt.shape[1]
    b = 1000

    def body(c_ref, o_ref):
        o_ref[...] = c_ref[0, :, 0:1] + c_ref[1, :, 0:1]

    return pl.pallas_call(
        body,
        grid=(n // b,),
        in_specs=[pl.BlockSpec((NC, b, D), lambda i: (0, i, 0))],
        out_specs=pl.BlockSpec((b, 1), lambda i: (i, 0)),
        out_shape=jax.ShapeDtypeStruct((n, 1), jnp.float32),
    )(cnt)


def _combine(seg, cnt, xdst, Wl, bl, Wr):
    n = xdst.shape[0]
    b = 1000

    def body(seg_ref, cnt_ref, x_ref, wl_ref, bl_ref, wr_ref, o_ref):
        segs = seg_ref[0] + seg_ref[1]
        mean = segs / jnp.maximum(cnt_ref[...], 1.0)
        y = (jnp.dot(mean, wl_ref[...], preferred_element_type=jnp.float32)
             + bl_ref[...]
             + jnp.dot(x_ref[...], wr_ref[...],
                       preferred_element_type=jnp.float32))
        o_ref[...] = jnp.where(y >= 0.0, y, 0.1 * y)

    return pl.pallas_call(
        body,
        grid=(n // b,),
        in_specs=[
            pl.BlockSpec((NC, b, D), lambda i: (0, i, 0)),
            pl.BlockSpec((b, 1), lambda i: (i, 0)),
            pl.BlockSpec((b, D), lambda i: (i, 0)),
            pl.BlockSpec((D, D), lambda i: (0, 0)),
            pl.BlockSpec((1, D), lambda i: (0, 0)),
            pl.BlockSpec((D, D), lambda i: (0, 0)),
        ],
        out_specs=pl.BlockSpec((b, D), lambda i: (i, 0)),
        out_shape=jax.ShapeDtypeStruct((n, D), jnp.float32),
    )(seg, cnt, xdst, Wl, bl.reshape(1, D), Wr)


def _combine_final(seg, cnt, xdst, Wl, bl, Wr, lin_W, lin_b):
    n = xdst.shape[0]
    b = 1000

    def body(seg_ref, cnt_ref, x_ref, wl_ref, bl_ref, wr_ref, lw_ref, lb_ref,
             o_ref):
        segs = seg_ref[0] + seg_ref[1]
        mean = segs / jnp.maximum(cnt_ref[...], 1.0)
        y = (jnp.dot(mean, wl_ref[...], preferred_element_type=jnp.float32)
             + bl_ref[...]
             + jnp.dot(x_ref[...], wr_ref[...],
                       preferred_element_type=jnp.float32))
        y = jnp.where(y >= 0.0, y, 0.1 * y)
        o_ref[...] = (jnp.dot(y, lw_ref[...],
                              preferred_element_type=jnp.float32)
                      + lb_ref[...])

    return pl.pallas_call(
        body,
        grid=(n // b,),
        in_specs=[
            pl.BlockSpec((NC, b, D), lambda i: (0, i, 0)),
            pl.BlockSpec((b, 1), lambda i: (i, 0)),
            pl.BlockSpec((b, D), lambda i: (i, 0)),
            pl.BlockSpec((D, D), lambda i: (0, 0)),
            pl.BlockSpec((1, D), lambda i: (0, 0)),
            pl.BlockSpec((D, D), lambda i: (0, 0)),
            pl.BlockSpec((D, 1), lambda i: (0, 0)),
            pl.BlockSpec((1, 1), lambda i: (0, 0)),
        ],
        out_specs=pl.BlockSpec((b, 1), lambda i: (i, 0)),
        out_shape=jax.ShapeDtypeStruct((n, 1), jnp.float32),
    )(seg, cnt, xdst, Wl, bl.reshape(1, D), Wr, lin_W,
      lin_b.reshape(1, 1))


def kernel(x_feat, x_agg, ei_c2a_src, ei_c2a_dst, ei_a2c_src, ei_a2c_dst,
           enc_W, enc_b, l1_c2a_Wl, l1_c2a_bl, l1_c2a_Wr, l1_a2c_Wl,
           l1_a2c_bl, l1_a2c_Wr, l2_c2a_Wl, l2_c2a_bl, l2_c2a_Wr, l2_a2c_Wl,
           l2_a2c_bl, l2_a2c_Wr, lin_W, lin_b):
    clients0 = _encoder(x_feat, enc_W, enc_b)
    cntA, cntC = _sc_counts(ei_c2a_dst, ei_a2c_dst)
    segA, segC = _sc_layer1(
        clients0, x_agg, ei_c2a_src, ei_c2a_dst, ei_a2c_src, ei_a2c_dst)
    cntA_r = _cnt_reduce(cntA)
    cntC_r = _cnt_reduce(cntC)
    aggs1 = _combine(segA, cntA_r, x_agg, l1_c2a_Wl, l1_c2a_bl, l1_c2a_Wr)
    clients1 = _combine(segC, cntC_r, clients0, l1_a2c_Wl, l1_a2c_bl,
                        l1_a2c_Wr)
    segC2 = _sc_layer2(aggs1, ei_a2c_src, ei_a2c_dst)
    out = _combine_final(segC2, cntC_r, clients1, l2_a2c_Wl, l2_a2c_bl,
                         l2_a2c_Wr, lin_W, lin_b)
    return out[:, 0]


# speedup vs baseline: 9.4068x; 1.1387x over previous
import functools

import jax
import jax.numpy as jnp
from jax import lax
from jax.experimental import pallas as pl
from jax.experimental.pallas import tpu as pltpu
from jax.experimental.pallas import tpu_sc as plsc

N_CLIENTS = 10000
N_AGG = 1000
E = 320000
D = 128

NC = 2
NS = 16
NW = NC * NS
EPW = E // NW
CH = 80
NCHUNK = EPW // CH
RCH = 25
RND = NCHUNK // RCH

_mesh = plsc.VectorSubcoreMesh(core_axis_name="c", subcore_axis_name="s")


def _fill_vmem2d(ref, val):
    rows, width = ref.shape
    v = jnp.full((16,), val, jnp.float32)

    def body(i, carry):
        for j in range(width // 16):
            ref[i, pl.ds(j * 16, 16)] = v
        return carry

    lax.fori_loop(0, rows, body, 0)


def _zero_shared(acc, zsrc, sid):
    n = acc.shape[0]
    zr = zsrc.shape[0]
    nwin = (n + zr - 1) // zr
    for w in range(nwin):
        sz = min(zr, n - w * zr)

        @pl.when(sid == (w % NS))
        def _():
            src = zsrc if sz == zr else zsrc.at[pl.ds(0, sz)]
            pltpu.sync_copy(src, acc.at[pl.ds(w * zr, sz)])


def _writeback(acc, out_hbm, bounce, core, sid):
    n = acc.shape[0]
    zr = bounce.shape[0]
    nwin = (n + zr - 1) // zr
    for w in range(nwin):
        sz = min(zr, n - w * zr)

        @pl.when(sid == (w % NS))
        def _():
            dst = bounce if sz == zr else bounce.at[pl.ds(0, sz)]
            pltpu.sync_copy(acc.at[pl.ds(w * zr, sz)], dst)
            pltpu.sync_copy(dst, out_hbm.at[core, pl.ds(w * zr, sz)])


def _edge_pass(table_hbm, src_hbm, dst_hbm, acc, idx, rows, semg, semi, wid):
    base = wid * EPW
    (sidx_a, didx_a), (sidx_b, didx_b), (sidx_c, didx_c) = idx
    rows_a, rows_b, rows_c = rows
    semg_a, semg_b, semg_c = semg
    semi_a, semi_b, semi_c = semi

    def _load_idx(c, sv, dv, sem):
        off = base + jnp.minimum(c, NCHUNK - 1) * CH
        pltpu.async_copy(src_hbm.at[pl.ds(off, CH)], sv, sem)
        pltpu.async_copy(dst_hbm.at[pl.ds(off, CH)], dv, sem)

    def _drain_idx(sv, dv, sem):
        pltpu.make_async_copy(src_hbm.at[pl.ds(0, CH)], sv, sem).wait()
        pltpu.make_async_copy(src_hbm.at[pl.ds(0, CH)], dv, sem).wait()

    def _gather(sv, buf, sem):
        pltpu.async_copy(table_hbm.at[sv], buf, sem)

    def _drain_g(buf, sem):
        pltpu.make_async_copy(table_hbm.at[sidx_a], buf, sem).wait()

    def _scatter(dv, buf):
        pltpu.sync_copy(buf, acc.at[dv], add=True)

    _load_idx(0, sidx_a, didx_a, semi_a)
    _load_idx(1, sidx_b, didx_b, semi_b)
    _drain_idx(sidx_a, didx_a, semi_a)
    _gather(sidx_a, rows_a, semg_a)
    _drain_idx(sidx_b, didx_b, semi_b)
    _gather(sidx_b, rows_b, semg_b)
    _load_idx(2, sidx_c, didx_c, semi_c)

    def triple(k, carry):
        c0 = 3 * k
        _drain_idx(sidx_c, didx_c, semi_c)
        _gather(sidx_c, rows_c, semg_c)
        _drain_g(rows_a, semg_a)
        _scatter(didx_a, rows_a)
        _load_idx(c0 + 3, sidx_a, didx_a, semi_a)
        _drain_g(rows_b, semg_b)
        _scatter(didx_b, rows_b)
        _load_idx(c0 + 4, sidx_b, didx_b, semi_b)
        _drain_idx(sidx_a, didx_a, semi_a)
        _gather(sidx_a, rows_a, semg_a)
        _drain_g(rows_c, semg_c)
        _scatter(didx_c, rows_c)
        _load_idx(c0 + 5, sidx_c, didx_c, semi_c)
        _drain_idx(sidx_b, didx_b, semi_b)
        _gather(sidx_b, rows_b, semg_b)
        return carry

    lax.fori_loop(0, (NCHUNK - 2) // 3, triple, 0)
    _drain_g(rows_a, semg_a)
    _scatter(didx_a, rows_a)
    _drain_g(rows_b, semg_b)
    _scatter(didx_b, rows_b)
    _drain_idx(sidx_c, didx_c, semi_c)


def _count_pass(dst_hbm, cacc, didx_a, didx_b, ones_b, semi_a, semi_b, wid):
    base = wid * EPW

    def _load_idx(c, dv, sem):
        off = base + jnp.minimum(c, NCHUNK - 1) * CH
        pltpu.async_copy(dst_hbm.at[pl.ds(off, CH)], dv, sem)

    def _drain_idx(dv, sem):
        pltpu.make_async_copy(dst_hbm.at[pl.ds(0, CH)], dv, sem).wait()

    _load_idx(0, didx_a, semi_a)
    _drain_idx(didx_a, semi_a)
    _load_idx(1, didx_b, semi_b)

    def pair(k, carry):
        c0 = 2 * k
        pltpu.sync_copy(ones_b, cacc.at[didx_a], add=True)
        _load_idx(c0 + 2, didx_a, semi_a)
        _drain_idx(didx_b, semi_b)
        pltpu.sync_copy(ones_b, cacc.at[didx_b], add=True)
        _load_idx(c0 + 3, didx_b, semi_b)
        _drain_idx(didx_a, semi_a)
        return carry

    lax.fori_loop(0, (NCHUNK - 1) // 2, pair, 0)
    _drain_idx(didx_b, semi_b)
    pltpu.sync_copy(ones_b, cacc.at[didx_a], add=True)


@functools.partial(
    pl.kernel,
    out_type=(
        jax.ShapeDtypeStruct((NC, N_AGG, D), jnp.float32),
        jax.ShapeDtypeStruct((NC, N_CLIENTS, D), jnp.float32),
    ),
    mesh=_mesh,
    scratch_types=[
        pltpu.VMEM((CH,), jnp.int32),
        pltpu.VMEM((CH,), jnp.int32),
        pltpu.VMEM((CH, D), jnp.float32),
        pltpu.VMEM((CH, D), jnp.float32),
        pltpu.VMEM_SHARED((N_AGG, D), jnp.float32),
        pltpu.VMEM_SHARED((N_CLIENTS, D), jnp.float32),
        pltpu.SemaphoreType.DMA,
        pltpu.SemaphoreType.DMA,
    ],
)
def _sc_counts(c2a_dst, a2c_dst, cntA_hbm, cntC_hbm,
               didx_a, didx_b, ones_b, zrows, caccA, caccC,
               semi_a, semi_b):
    core = lax.axis_index("c")
    sid = lax.axis_index("s")
    wid = core * NS + sid

    _fill_vmem2d(ones_b, 1.0)
    _fill_vmem2d(zrows, 0.0)
    _zero_shared(caccA, zrows, sid)
    _zero_shared(caccC, zrows, sid)
    plsc.subcore_barrier()

    _count_pass(c2a_dst, caccA, didx_a, didx_b, ones_b, semi_a, semi_b, wid)
    _count_pass(a2c_dst, caccC, didx_a, didx_b, ones_b, semi_a, semi_b, wid)
    plsc.subcore_barrier()

    _writeback(caccA, cntA_hbm, zrows, core, sid)
    _writeback(caccC, cntC_hbm, zrows, core, sid)


@functools.partial(
    pl.kernel,
    out_type=(
        jax.ShapeDtypeStruct((NC, N_AGG, D), jnp.float32),
        jax.ShapeDtypeStruct((NC, N_CLIENTS, D), jnp.float32),
    ),
    mesh=_mesh,
    scratch_types=[
        [[(pltpu.VMEM((CH,), jnp.int32), pltpu.VMEM((CH,), jnp.int32))
          for _ in range(3)]],
        [pltpu.VMEM((CH, D), jnp.float32) for _ in range(3)],
        pltpu.VMEM_SHARED((N_AGG, D), jnp.float32),
        pltpu.VMEM_SHARED((N_CLIENTS, D), jnp.float32),
        [pltpu.SemaphoreType.DMA for _ in range(3)],
        [pltpu.SemaphoreType.DMA for _ in range(3)],
    ],
)
def _sc_layer1(clients_hbm, aggs_hbm, c2a_src, c2a_dst, a2c_src, a2c_dst,
               segA_hbm, segC_hbm,
               idx, rows, accA, accC, semg, semi):
    core = lax.axis_index("c")
    sid = lax.axis_index("s")
    wid = core * NS + sid

    _fill_vmem2d(rows[0], 0.0)
    _zero_shared(accA, rows[0], sid)
    _zero_shared(accC, rows[0], sid)
    plsc.subcore_barrier()

    _edge_pass(clients_hbm, c2a_src, c2a_dst, accA, idx[0], rows, semg,
               semi, wid)
    _edge_pass(aggs_hbm, a2c_src, a2c_dst, accC, idx[0], rows, semg,
               semi, wid)
    plsc.subcore_barrier()

    _writeback(accA, segA_hbm, rows[0], core, sid)
    _writeback(accC, segC_hbm, rows[0], core, sid)


@functools.partial(
    pl.kernel,
    out_type=jax.ShapeDtypeStruct((NC, N_CLIENTS, D), jnp.float32),
    mesh=_mesh,
    scratch_types=[
        [[(pltpu.VMEM((CH,), jnp.int32), pltpu.VMEM((CH,), jnp.int32))
          for _ in range(3)]],
        [pltpu.VMEM((CH, D), jnp.float32) for _ in range(3)],
        pltpu.VMEM_SHARED((N_CLIENTS, D), jnp.float32),
        [pltpu.SemaphoreType.DMA for _ in range(3)],
        [pltpu.SemaphoreType.DMA for _ in range(3)],
    ],
)
def _sc_layer2(aggs_hbm, a2c_src, a2c_dst, segC_hbm,
               idx, rows, accC, semg, semi):
    core = lax.axis_index("c")
    sid = lax.axis_index("s")
    wid = core * NS + sid

    _fill_vmem2d(rows[0], 0.0)
    _zero_shared(accC, rows[0], sid)
    plsc.subcore_barrier()

    _edge_pass(aggs_hbm, a2c_src, a2c_dst, accC, idx[0], rows, semg,
               semi, wid)
    plsc.subcore_barrier()

    _writeback(accC, segC_hbm, rows[0], core, sid)


def _encoder(x_feat, enc_W, enc_b):
    n = x_feat.shape[0]
    b = 1000

    def body(x_ref, w_ref, b_ref, o_ref):
        o_ref[...] = (jnp.dot(x_ref[...], w_ref[...],
                              preferred_element_type=jnp.float32)
                      + b_ref[...]) * 0.5

    return pl.pallas_call(
        body,
        grid=(n // b,),
        in_specs=[
            pl.BlockSpec((b, D), lambda i: (i, 0)),
            pl.BlockSpec((D, D), lambda i: (0, 0)),
            pl.BlockSpec((1, D), lambda i: (0, 0)),
        ],
        out_specs=pl.BlockSpec((b, D), lambda i: (i, 0)),
        out_shape=jax.ShapeDtypeStruct((n, D), jnp.float32),
    )(x_feat, enc_W, enc_b.reshape(1, D))


def _cnt_reduce(cnt):
    n = cnt.shape[1]
    b = 1000

    def body(c_ref, o_ref):
        o_ref[...] = c_ref[0, :, 0:1] + c_ref[1, :, 0:1]

    return pl.pallas_call(
        body,
        grid=(n // b,),
        in_specs=[pl.BlockSpec((NC, b, D), lambda i: (0, i, 0))],
        out_specs=pl.BlockSpec((b, 1), lambda i: (i, 0)),
        out_shape=jax.ShapeDtypeStruct((n, 1), jnp.float32),
    )(cnt)


def _combine(seg, cnt, xdst, Wl, bl, Wr):
    n = xdst.shape[0]
    b = 1000

    def body(seg_ref, cnt_ref, x_ref, wl_ref, bl_ref, wr_ref, o_ref):
        segs = seg_ref[0] + seg_ref[1]
        mean = segs / jnp.maximum(cnt_ref[...], 1.0)
        y = (jnp.dot(mean, wl_ref[...], preferred_element_type=jnp.float32)
             + bl_ref[...]
             + jnp.dot(x_ref[...], wr_ref[...],
                       preferred_element_type=jnp.float32))
        o_ref[...] = jnp.where(y >= 0.0, y, 0.1 * y)

    return pl.pallas_call(
        body,
        grid=(n // b,),
        in_specs=[
            pl.BlockSpec((NC, b, D), lambda i: (0, i, 0)),
            pl.BlockSpec((b, 1), lambda i: (i, 0)),
            pl.BlockSpec((b, D), lambda i: (i, 0)),
            pl.BlockSpec((D, D), lambda i: (0, 0)),
            pl.BlockSpec((1, D), lambda i: (0, 0)),
            pl.BlockSpec((D, D), lambda i: (0, 0)),
        ],
        out_specs=pl.BlockSpec((b, D), lambda i: (i, 0)),
        out_shape=jax.ShapeDtypeStruct((n, D), jnp.float32),
    )(seg, cnt, xdst, Wl, bl.reshape(1, D), Wr)


def _combine_final(seg, cnt, xdst, Wl, bl, Wr, lin_W, lin_b):
    n = xdst.shape[0]
    b = 1000

    def body(seg_ref, cnt_ref, x_ref, wl_ref, bl_ref, wr_ref, lw_ref, lb_ref,
             o_ref):
        segs = seg_ref[0] + seg_ref[1]
        mean = segs / jnp.maximum(cnt_ref[...], 1.0)
        y = (jnp.dot(mean, wl_ref[...], preferred_element_type=jnp.float32)
             + bl_ref[...]
             + jnp.dot(x_ref[...], wr_ref[...],
                       preferred_element_type=jnp.float32))
        y = jnp.where(y >= 0.0, y, 0.1 * y)
        o_ref[...] = (jnp.dot(y, lw_ref[...],
                              preferred_element_type=jnp.float32)
                      + lb_ref[...])

    return pl.pallas_call(
        body,
        grid=(n // b,),
        in_specs=[
            pl.BlockSpec((NC, b, D), lambda i: (0, i, 0)),
            pl.BlockSpec((b, 1), lambda i: (i, 0)),
            pl.BlockSpec((b, D), lambda i: (i, 0)),
            pl.BlockSpec((D, D), lambda i: (0, 0)),
            pl.BlockSpec((1, D), lambda i: (0, 0)),
            pl.BlockSpec((D, D), lambda i: (0, 0)),
            pl.BlockSpec((D, 1), lambda i: (0, 0)),
            pl.BlockSpec((1, 1), lambda i: (0, 0)),
        ],
        out_specs=pl.BlockSpec((b, 1), lambda i: (i, 0)),
        out_shape=jax.ShapeDtypeStruct((n, 1), jnp.float32),
    )(seg, cnt, xdst, Wl, bl.reshape(1, D), Wr, lin_W,
      lin_b.reshape(1, 1))


def kernel(x_feat, x_agg, ei_c2a_src, ei_c2a_dst, ei_a2c_src, ei_a2c_dst,
           enc_W, enc_b, l1_c2a_Wl, l1_c2a_bl, l1_c2a_Wr, l1_a2c_Wl,
           l1_a2c_bl, l1_a2c_Wr, l2_c2a_Wl, l2_c2a_bl, l2_c2a_Wr, l2_a2c_Wl,
           l2_a2c_bl, l2_a2c_Wr, lin_W, lin_b):
    clients0 = _encoder(x_feat, enc_W, enc_b)
    cntA, cntC = _sc_counts(ei_c2a_dst, ei_a2c_dst)
    segA, segC = _sc_layer1(
        clients0, x_agg, ei_c2a_src, ei_c2a_dst, ei_a2c_src, ei_a2c_dst)
    cntA_r = _cnt_reduce(cntA)
    cntC_r = _cnt_reduce(cntC)
    aggs1 = _combine(segA, cntA_r, x_agg, l1_c2a_Wl, l1_c2a_bl, l1_c2a_Wr)
    clients1 = _combine(segC, cntC_r, clients0, l1_a2c_Wl, l1_a2c_bl,
                        l1_a2c_Wr)
    segC2 = _sc_layer2(aggs1, ei_a2c_src, ei_a2c_dst)
    out = _combine_final(segC2, cntC_r, clients1, l2_a2c_Wl, l2_a2c_bl,
                         l2_a2c_Wr, lin_W, lin_b)
    return out[:, 0]
